# baseline (speedup 1.0000x reference)
import functools

import jax
import jax.numpy as jnp
from jax import lax
from jax.experimental import pallas as pl
from jax.experimental.pallas import tpu as pltpu
from jax.experimental.pallas import tpu_sc as plsc

T = 8192
D_MODEL = 1024
D_OUT = 1024
N_LEAF = 16
N_INT = 15

BT_A = 512
NT_A = T // BT_A
BT_B = 512
NT_B = T // BT_B
BT_MM = 128
S = T + N_LEAF * BT_MM
NT_MM = S // BT_MM
TL_PAD = 128


def _ab_body(x_ref, w_ref, b_ref, m_ref, tri_ref, pos_ref, tl_ref, leafbuf, scr):
    p = pl.program_id(0)
    i = pl.program_id(1)

    @pl.when(p == 0)
    def _():
        x = x_ref[...]
        logits = lax.dot_general(x, w_ref[...], (((1,), (1,)), ((), ())),
                                 preferred_element_type=jnp.float32) + b_ref[...]
        lp = jnp.concatenate(
            [jax.nn.log_sigmoid(-logits), jax.nn.log_sigmoid(logits)], axis=1)
        log_probs = lax.dot_general(lp, m_ref[...], (((1,), (0,)), ((), ())),
                                    preferred_element_type=jnp.float32)
        col = lax.broadcasted_iota(jnp.int32, log_probs.shape, 1)
        mx = jnp.max(log_probs, axis=1, keepdims=True)
        leaf = jnp.min(jnp.where(log_probs == mx, col, N_LEAF), axis=1)
        leafbuf[pl.ds(i, 1), :] = leaf.reshape(1, BT_A)
        onehot = (leaf[:, None] == lax.broadcasted_iota(
            jnp.int32, (BT_A, N_LEAF), 1)).astype(jnp.float32)
        tile_cnt = jnp.sum(onehot, axis=0, keepdims=True).astype(jnp.int32)

        @pl.when(i == 0)
        def _():
            scr[...] = jnp.zeros_like(scr)

        scr[0:1, 0:N_LEAF] = scr[0:1, 0:N_LEAF] + tile_cnt

    @pl.when((p == 1) & (i == 0))
    def _():
        cnt = scr[0:1, 0:N_LEAF]
        padded = ((cnt + BT_MM - 1) // BT_MM) * BT_MM
        pb = jnp.broadcast_to(padded, (N_LEAF, N_LEAF))
        rowj = lax.broadcasted_iota(jnp.int32, (N_LEAF, N_LEAF), 0)
        colk = lax.broadcasted_iota(jnp.int32, (N_LEAF, N_LEAF), 1)
        gs = jnp.sum(jnp.where(colk < rowj, pb, 0), axis=1).reshape(1, N_LEAF)
        scr[1:2, 0:N_LEAF] = gs
        ends = gs + padded
        rr = lax.broadcasted_iota(jnp.int32, (TL_PAD, N_LEAF), 0) * BT_MM
        tl = jnp.sum((rr >= jnp.broadcast_to(ends, (TL_PAD, N_LEAF))
                      ).astype(jnp.int32), axis=1)
        tl_ref[0, 0, :] = jnp.minimum(tl, N_LEAF - 1)
        scr[0:1, 0:N_LEAF] = jnp.zeros((1, N_LEAF), jnp.int32)

    @pl.when(p == 1)
    def _():
        lf = leafbuf[pl.ds(i, 1), :].reshape(BT_B)
        onehot = (lf[:, None] == lax.broadcasted_iota(
            jnp.int32, (BT_B, N_LEAF), 1)).astype(jnp.float32)
        tile_cnt = jnp.sum(onehot, axis=0, keepdims=True).astype(jnp.int32)
        prev = scr[0:1, 0:N_LEAF]
        gs = scr[1:2, 0:N_LEAF]
        rank = lax.dot_general(tri_ref[...], onehot, (((1,), (0,)), ((), ())),
                               preferred_element_type=jnp.float32)
        basef = (gs + prev).astype(jnp.float32)
        posv = jnp.sum(onehot * (basef + rank), axis=1)
        pos_ref[0, 0, :] = posv.astype(jnp.int32)
        scr[0:1, 0:N_LEAF] = prev + tile_cnt


def _ab(xs, weights, b2, mperm, tri):
    return pl.pallas_call(
        _ab_body,
        grid=(2, NT_B),
        in_specs=[
            pl.BlockSpec((BT_A, D_MODEL),
                         lambda p, i: (jnp.where(p == 0, i, 0), 0)),
            pl.BlockSpec((N_INT, D_MODEL), lambda p, i: (0, 0)),
            pl.BlockSpec((1, N_INT), lambda p, i: (0, 0)),
            pl.BlockSpec((2 * N_INT, N_LEAF), lambda p, i: (0, 0)),
            pl.BlockSpec((BT_B, BT_B), lambda p, i: (0, 0)),
        ],
        out_specs=[
            pl.BlockSpec((1, 1, BT_B), lambda p, i: (i, 0, 0)),
            pl.BlockSpec((1, 1, TL_PAD), lambda p, i: (0, 0, 0)),
        ],
        out_shape=[
            jax.ShapeDtypeStruct((NT_B, 1, BT_B), jnp.int32),
            jax.ShapeDtypeStruct((1, 1, TL_PAD), jnp.int32),
        ],
        scratch_shapes=[pltpu.VMEM((NT_B, BT_B), jnp.int32),
                        pltpu.VMEM((8, 128), jnp.int32)],
    )(xs, weights, b2, mperm, tri)


_NC = 2
_NS = 16
_NW = _NC * _NS
_PER_W = T // _NW
_SC_CHUNK = 64
_NCH = _PER_W // _SC_CHUNK


def _sc_permute(src, pos, out_rows, scatter):
    mesh = plsc.VectorSubcoreMesh(core_axis_name="c", subcore_axis_name="s")

    @functools.partial(
        pl.kernel, mesh=mesh,
        out_type=jax.ShapeDtypeStruct((out_rows, D_OUT), jnp.float32),
        scratch_types=(
            [pltpu.VMEM((_SC_CHUNK,), jnp.int32) for _ in range(_NCH)]
            + [pltpu.VMEM((_SC_CHUNK, D_OUT), jnp.float32),
               pltpu.SemaphoreType.DMA]),
    )
    def k(src_hbm, pos_hbm, out_hbm, i0, i1, i2, i3, rows_v, sem):
        wid = lax.axis_index("s") * _NC + lax.axis_index("c")
        base = wid * _PER_W
        idx = [i0, i1, i2, i3]
        for c in range(_NCH):
            pltpu.sync_copy(pos_hbm.at[pl.ds(base + c * _SC_CHUNK, _SC_CHUNK)],
                            idx[c])
        for c in range(_NCH):
            lin = pl.ds(base + c * _SC_CHUNK, _SC_CHUNK)
            if scatter:
                pltpu.sync_copy(src_hbm.at[lin], rows_v)
                pltpu.async_copy(rows_v, out_hbm.at[idx[c]], sem).wait()
            else:
                pltpu.async_copy(src_hbm.at[idx[c]], rows_v, sem).wait()
                pltpu.sync_copy(rows_v, out_hbm.at[lin])

    return k(src, pos)


def _mm_body(tl_ref, x_ref, w_ref, b_ref, o_ref):
    del tl_ref
    o_ref[...] = lax.dot_general(
        x_ref[...], w_ref[0], (((1,), (0,)), ((), ())),
        preferred_element_type=jnp.float32) + b_ref[0]


def _grouped_mm(tile_leaf, xs_sorted, leaf_W, leaf_b):
    grid_spec = pltpu.PrefetchScalarGridSpec(
        num_scalar_prefetch=1,
        grid=(NT_MM,),
        in_specs=[
            pl.BlockSpec((BT_MM, D_MODEL), lambda i, tl: (i, 0)),
            pl.BlockSpec((1, D_MODEL, D_OUT), lambda i, tl: (tl[i], 0, 0)),
            pl.BlockSpec((1, 1, D_OUT), lambda i, tl: (tl[i], 0, 0)),
        ],
        out_specs=pl.BlockSpec((BT_MM, D_OUT), lambda i, tl: (i, 0)),
    )
    return pl.pallas_call(
        _mm_body,
        grid_spec=grid_spec,
        out_shape=jax.ShapeDtypeStruct((S, D_OUT), jnp.float32),
    )(tile_leaf, xs_sorted, leaf_W, leaf_b.reshape(N_LEAF, 1, D_OUT))


def kernel(xs, weights, biases, masks, leaf_W, leaf_b):
    mperm = jnp.concatenate([masks[:, 0::2], masks[:, 1::2]], axis=1).T
    b2 = biases.reshape(1, N_INT)
    tri = jnp.tril(jnp.ones((BT_B, BT_B), jnp.float32), -1)
    pos3, tl3 = _ab(xs, weights, b2, mperm, tri)
    pos = pos3.reshape(T)
    tile_leaf = tl3.reshape(TL_PAD)
    xs_sorted = _sc_permute(xs, pos, S, scatter=True)
    out_sorted = _grouped_mm(tile_leaf, xs_sorted, leaf_W, leaf_b)
    return _sc_permute(out_sorted, pos, T, scatter=False)


# speedup vs baseline: 1.1409x; 1.1409x over previous
import functools

import jax
import jax.numpy as jnp
from jax import lax
from jax.experimental import pallas as pl
from jax.experimental.pallas import tpu as pltpu
from jax.experimental.pallas import tpu_sc as plsc

T = 8192
D_MODEL = 1024
D_OUT = 1024
N_LEAF = 16
N_INT = 15

BT_A = 512
NT_A = T // BT_A
BT_B = 512
NT_B = T // BT_B
BT_MM = 256
S = T + N_LEAF * BT_MM
NT_MM = S // BT_MM
TL_PAD = 128


def _ab_body(x_ref, w_ref, b_ref, m_ref, tri_ref, pos_ref, tl_ref, leafbuf, scr):
    p = pl.program_id(0)
    i = pl.program_id(1)

    @pl.when(p == 0)
    def _():
        x = x_ref[...]
        logits = lax.dot_general(w_ref[...], x, (((1,), (1,)), ((), ())),
                                 preferred_element_type=jnp.float32) + b_ref[...]
        lp = jnp.concatenate(
            [jax.nn.log_sigmoid(-logits), jax.nn.log_sigmoid(logits)], axis=0)
        log_probs = lax.dot_general(m_ref[...], lp, (((1,), (0,)), ((), ())),
                                    preferred_element_type=jnp.float32)
        row = lax.broadcasted_iota(jnp.int32, (N_LEAF, BT_A), 0)
        mx = jnp.max(log_probs, axis=0, keepdims=True)
        leaf = jnp.min(jnp.where(log_probs == mx, row, N_LEAF), axis=0,
                       keepdims=True)
        leafbuf[pl.ds(i, 1), :] = leaf
        onehot = (leaf == row).astype(jnp.float32)
        tile_cnt = jnp.sum(onehot, axis=1, keepdims=True).astype(jnp.int32)

        @pl.when(i == 0)
        def _():
            scr[...] = jnp.zeros_like(scr)

        scr[0:N_LEAF, 0:1] = scr[0:N_LEAF, 0:1] + tile_cnt

    @pl.when((p == 1) & (i == 0))
    def _():
        cnt = scr[0:N_LEAF, 0:1]
        padded = ((cnt + BT_MM - 1) // BT_MM) * BT_MM
        rowj = lax.broadcasted_iota(jnp.int32, (N_LEAF, N_LEAF), 0)
        colk = lax.broadcasted_iota(jnp.int32, (N_LEAF, N_LEAF), 1)
        low = (colk < rowj).astype(jnp.float32)
        gs = lax.dot_general(low, padded.astype(jnp.float32),
                             (((1,), (0,)), ((), ())),
                             preferred_element_type=jnp.float32
                             ).astype(jnp.int32)
        scr[0:N_LEAF, 1:2] = gs
        ends = gs + padded
        rr = lax.broadcasted_iota(jnp.int32, (N_LEAF, TL_PAD), 1) * BT_MM
        tl = jnp.sum((rr >= ends).astype(jnp.int32), axis=0, keepdims=True)
        tl_ref[0, 0, :] = jnp.minimum(tl, N_LEAF - 1)[0]
        scr[0:N_LEAF, 0:1] = jnp.zeros((N_LEAF, 1), jnp.int32)

    @pl.when(p == 1)
    def _():
        lf = leafbuf[pl.ds(i, 1), :]
        row = lax.broadcasted_iota(jnp.int32, (N_LEAF, BT_B), 0)
        onehot = (lf == row).astype(jnp.float32)
        tile_cnt = jnp.sum(onehot, axis=1, keepdims=True).astype(jnp.int32)
        prev = scr[0:N_LEAF, 0:1]
        gs = scr[0:N_LEAF, 1:2]
        rank = lax.dot_general(onehot, tri_ref[...], (((1,), (0,)), ((), ())),
                               preferred_element_type=jnp.float32)
        basef = (gs + prev).astype(jnp.float32)
        posv = jnp.sum(onehot * (basef + rank), axis=0, keepdims=True)
        pos_ref[0, 0, :] = posv.astype(jnp.int32)[0]
        scr[0:N_LEAF, 0:1] = prev + tile_cnt


def _ab(xs, weights, b2, mperm, tri):
    return pl.pallas_call(
        _ab_body,
        grid=(2, NT_B),
        in_specs=[
            pl.BlockSpec((BT_A, D_MODEL),
                         lambda p, i: (jnp.where(p == 0, i, 0), 0)),
            pl.BlockSpec((N_INT, D_MODEL), lambda p, i: (0, 0)),
            pl.BlockSpec((N_INT, 1), lambda p, i: (0, 0)),
            pl.BlockSpec((N_LEAF, 2 * N_INT), lambda p, i: (0, 0)),
            pl.BlockSpec((BT_B, BT_B), lambda p, i: (0, 0)),
        ],
        out_specs=[
            pl.BlockSpec((1, 1, BT_B), lambda p, i: (i, 0, 0)),
            pl.BlockSpec((1, 1, TL_PAD), lambda p, i: (0, 0, 0)),
        ],
        out_shape=[
            jax.ShapeDtypeStruct((NT_B, 1, BT_B), jnp.int32),
            jax.ShapeDtypeStruct((1, 1, TL_PAD), jnp.int32),
        ],
        scratch_shapes=[pltpu.VMEM((NT_B, BT_B), jnp.int32),
                        pltpu.VMEM((N_LEAF, 128), jnp.int32)],
    )(xs, weights, b2, mperm, tri)


_NC = 2
_NS = 16
_NW = _NC * _NS
_PER_W = T // _NW
_SC_CHUNK = 64
_NCH = _PER_W // _SC_CHUNK


def _sc_permute(src, pos, out_rows, scatter):
    mesh = plsc.VectorSubcoreMesh(core_axis_name="c", subcore_axis_name="s")

    @functools.partial(
        pl.kernel, mesh=mesh,
        out_type=jax.ShapeDtypeStruct((out_rows, D_OUT), jnp.float32),
        scratch_types=(
            [pltpu.VMEM((_SC_CHUNK,), jnp.int32) for _ in range(_NCH)]
            + [pltpu.VMEM((_SC_CHUNK, D_OUT), jnp.float32),
               pltpu.SemaphoreType.DMA]),
    )
    def k(src_hbm, pos_hbm, out_hbm, i0, i1, i2, i3, rows_v, sem):
        wid = lax.axis_index("s") * _NC + lax.axis_index("c")
        base = wid * _PER_W
        idx = [i0, i1, i2, i3]
        for c in range(_NCH):
            pltpu.sync_copy(pos_hbm.at[pl.ds(base + c * _SC_CHUNK, _SC_CHUNK)],
                            idx[c])
        for c in range(_NCH):
            lin = pl.ds(base + c * _SC_CHUNK, _SC_CHUNK)
            if scatter:
                pltpu.sync_copy(src_hbm.at[lin], rows_v)
                pltpu.async_copy(rows_v, out_hbm.at[idx[c]], sem).wait()
            else:
                pltpu.async_copy(src_hbm.at[idx[c]], rows_v, sem).wait()
                pltpu.sync_copy(rows_v, out_hbm.at[lin])

    return k(src, pos)


def _mm_body(tl_ref, x_ref, w_ref, b_ref, o_ref):
    del tl_ref
    o_ref[...] = lax.dot_general(
        x_ref[...], w_ref[0], (((1,), (0,)), ((), ())),
        preferred_element_type=jnp.float32) + b_ref[0]


def _grouped_mm(tile_leaf, xs_sorted, leaf_W, leaf_b):
    grid_spec = pltpu.PrefetchScalarGridSpec(
        num_scalar_prefetch=1,
        grid=(NT_MM,),
        in_specs=[
            pl.BlockSpec((BT_MM, D_MODEL), lambda i, tl: (i, 0)),
            pl.BlockSpec((1, D_MODEL, D_OUT), lambda i, tl: (tl[i], 0, 0)),
            pl.BlockSpec((1, 1, D_OUT), lambda i, tl: (tl[i], 0, 0)),
        ],
        out_specs=pl.BlockSpec((BT_MM, D_OUT), lambda i, tl: (i, 0)),
    )
    return pl.pallas_call(
        _mm_body,
        grid_spec=grid_spec,
        out_shape=jax.ShapeDtypeStruct((S, D_OUT), jnp.float32),
    )(tile_leaf, xs_sorted, leaf_W, leaf_b.reshape(N_LEAF, 1, D_OUT))


def kernel(xs, weights, biases, masks, leaf_W, leaf_b):
    mperm = jnp.concatenate([masks[:, 0::2], masks[:, 1::2]], axis=1)
    b2 = biases.reshape(N_INT, 1)
    tri = jnp.triu(jnp.ones((BT_B, BT_B), jnp.float32), 1)
    pos3, tl3 = _ab(xs, weights, b2, mperm, tri)
    pos = pos3.reshape(T)
    tile_leaf = tl3.reshape(TL_PAD)
    xs_sorted = _sc_permute(xs, pos, S, scatter=True)
    out_sorted = _grouped_mm(tile_leaf, xs_sorted, leaf_W, leaf_b)
    return _sc_permute(out_sorted, pos, T, scatter=False)


# speedup vs baseline: 1.1675x; 1.0232x over previous
import functools

import jax
import jax.numpy as jnp
from jax import lax
from jax.experimental import pallas as pl
from jax.experimental.pallas import tpu as pltpu
from jax.experimental.pallas import tpu_sc as plsc

T = 8192
D_MODEL = 1024
D_OUT = 1024
N_LEAF = 16
N_INT = 15

BT_A = 512
NT_A = T // BT_A
BT_B = 512
NT_B = T // BT_B
BT_MM = 256
S = T + N_LEAF * BT_MM
NT_MM = S // BT_MM
TL_PAD = 128


def _ab_body(x_ref, w_ref, b_ref, m_ref, tri_ref, pos_ref, tl_ref, leafbuf, scr):
    p = pl.program_id(0)
    i = pl.program_id(1)

    @pl.when(p == 0)
    def _():
        x = x_ref[...]
        logits = lax.dot_general(w_ref[...], x, (((1,), (1,)), ((), ())),
                                 preferred_element_type=jnp.float32) + b_ref[...]
        lp = jnp.concatenate(
            [jax.nn.log_sigmoid(-logits), jax.nn.log_sigmoid(logits)], axis=0)
        log_probs = lax.dot_general(m_ref[...], lp, (((1,), (0,)), ((), ())),
                                    preferred_element_type=jnp.float32)
        row = lax.broadcasted_iota(jnp.int32, (N_LEAF, BT_A), 0)
        mx = jnp.max(log_probs, axis=0, keepdims=True)
        leaf = jnp.min(jnp.where(log_probs == mx, row, N_LEAF), axis=0,
                       keepdims=True)
        leafbuf[pl.ds(i, 1), :] = leaf
        onehot = (leaf == row).astype(jnp.float32)
        tile_cnt = jnp.sum(onehot, axis=1, keepdims=True).astype(jnp.int32)

        @pl.when(i == 0)
        def _():
            scr[...] = jnp.zeros_like(scr)

        scr[0:N_LEAF, 0:1] = scr[0:N_LEAF, 0:1] + tile_cnt

    @pl.when((p == 1) & (i == 0))
    def _():
        cnt = scr[0:N_LEAF, 0:1]
        padded = ((cnt + BT_MM - 1) // BT_MM) * BT_MM
        rowj = lax.broadcasted_iota(jnp.int32, (N_LEAF, N_LEAF), 0)
        colk = lax.broadcasted_iota(jnp.int32, (N_LEAF, N_LEAF), 1)
        low = (colk < rowj).astype(jnp.float32)
        gs = lax.dot_general(low, padded.astype(jnp.float32),
                             (((1,), (0,)), ((), ())),
                             preferred_element_type=jnp.float32
                             ).astype(jnp.int32)
        scr[0:N_LEAF, 1:2] = gs
        ends = gs + padded
        rr = lax.broadcasted_iota(jnp.int32, (N_LEAF, TL_PAD), 1) * BT_MM
        tl = jnp.sum((rr >= ends).astype(jnp.int32), axis=0, keepdims=True)
        tl_ref[0, 0, :] = jnp.minimum(tl, N_LEAF - 1)[0]
        scr[0:N_LEAF, 0:1] = jnp.zeros((N_LEAF, 1), jnp.int32)

    @pl.when(p == 1)
    def _():
        lf = leafbuf[pl.ds(i, 1), :]
        row = lax.broadcasted_iota(jnp.int32, (N_LEAF, BT_B), 0)
        onehot = (lf == row).astype(jnp.float32)
        tile_cnt = jnp.sum(onehot, axis=1, keepdims=True).astype(jnp.int32)
        prev = scr[0:N_LEAF, 0:1]
        gs = scr[0:N_LEAF, 1:2]
        rank = lax.dot_general(onehot, tri_ref[...], (((1,), (0,)), ((), ())),
                               preferred_element_type=jnp.float32)
        basef = (gs + prev).astype(jnp.float32)
        posv = jnp.sum(onehot * (basef + rank), axis=0, keepdims=True)
        pos_ref[0, 0, :] = posv.astype(jnp.int32)[0]
        scr[0:N_LEAF, 0:1] = prev + tile_cnt


def _ab(xs, weights, b2, mperm, tri):
    return pl.pallas_call(
        _ab_body,
        grid=(2, NT_B),
        in_specs=[
            pl.BlockSpec((BT_A, D_MODEL),
                         lambda p, i: (jnp.where(p == 0, i, 0), 0)),
            pl.BlockSpec((N_INT, D_MODEL), lambda p, i: (0, 0)),
            pl.BlockSpec((N_INT, 1), lambda p, i: (0, 0)),
            pl.BlockSpec((N_LEAF, 2 * N_INT), lambda p, i: (0, 0)),
            pl.BlockSpec((BT_B, BT_B), lambda p, i: (0, 0)),
        ],
        out_specs=[
            pl.BlockSpec((1, 1, BT_B), lambda p, i: (i, 0, 0)),
            pl.BlockSpec((1, 1, TL_PAD), lambda p, i: (0, 0, 0)),
        ],
        out_shape=[
            jax.ShapeDtypeStruct((NT_B, 1, BT_B), jnp.int32),
            jax.ShapeDtypeStruct((1, 1, TL_PAD), jnp.int32),
        ],
        scratch_shapes=[pltpu.VMEM((NT_B, BT_B), jnp.int32),
                        pltpu.VMEM((N_LEAF, 128), jnp.int32)],
    )(xs, weights, b2, mperm, tri)


_NC = 2
_NS = 16
_NW = _NC * _NS
_PER_W = T // _NW
_SC_CHUNK = 32
_NCH = _PER_W // _SC_CHUNK


def _sc_permute(src, pos, out_rows, scatter):
    mesh = plsc.VectorSubcoreMesh(core_axis_name="c", subcore_axis_name="s")

    @functools.partial(
        pl.kernel, mesh=mesh,
        out_type=jax.ShapeDtypeStruct((out_rows, D_OUT), jnp.float32),
        scratch_types=(
            [pltpu.VMEM((_SC_CHUNK,), jnp.int32) for _ in range(_NCH)]
            + [pltpu.VMEM((_SC_CHUNK, D_OUT), jnp.float32),
               pltpu.VMEM((_SC_CHUNK, D_OUT), jnp.float32)]
            + [pltpu.SemaphoreType.DMA for _ in range(5)]),
    )
    def k(src_hbm, pos_hbm, out_hbm, i0, i1, i2, i3, i4, i5, i6, i7,
          b0, b1, sem_idx, si0, si1, so0, so1):
        wid = lax.axis_index("s") * _NC + lax.axis_index("c")
        base = wid * _PER_W
        idx = [i0, i1, i2, i3, i4, i5, i6, i7]
        bufs = [b0, b1]
        sin = [si0, si1]
        sout = [so0, so1]
        hidx = [pltpu.async_copy(
                    pos_hbm.at[pl.ds(base + c * _SC_CHUNK, _SC_CHUNK)],
                    idx[c], sem_idx) for c in range(_NCH)]
        for h in hidx:
            h.wait()

        def start_in(c):
            lin = pl.ds(base + c * _SC_CHUNK, _SC_CHUNK)
            if scatter:
                return pltpu.async_copy(src_hbm.at[lin], bufs[c % 2],
                                        sin[c % 2])
            return pltpu.async_copy(src_hbm.at[idx[c]], bufs[c % 2],
                                    sin[c % 2])

        def start_out(c):
            lin = pl.ds(base + c * _SC_CHUNK, _SC_CHUNK)
            if scatter:
                return pltpu.async_copy(bufs[c % 2], out_hbm.at[idx[c]],
                                        sout[c % 2])
            return pltpu.async_copy(bufs[c % 2], out_hbm.at[lin], sout[c % 2])

        ins = [None] * _NCH
        outs = [None] * _NCH
        ins[0] = start_in(0)
        ins[1] = start_in(1)
        for c in range(_NCH):
            ins[c].wait()
            outs[c] = start_out(c)
            if c + 2 < _NCH:
                outs[c].wait()
                ins[c + 2] = start_in(c + 2)
        outs[_NCH - 2].wait()
        outs[_NCH - 1].wait()

    return k(src, pos)


def _mm_body(tl_ref, x_ref, w_ref, b_ref, o_ref):
    del tl_ref
    o_ref[...] = lax.dot_general(
        x_ref[...], w_ref[0], (((1,), (0,)), ((), ())),
        preferred_element_type=jnp.float32) + b_ref[0]


def _grouped_mm(tile_leaf, xs_sorted, leaf_W, leaf_b):
    grid_spec = pltpu.PrefetchScalarGridSpec(
        num_scalar_prefetch=1,
        grid=(NT_MM,),
        in_specs=[
            pl.BlockSpec((BT_MM, D_MODEL), lambda i, tl: (i, 0)),
            pl.BlockSpec((1, D_MODEL, D_OUT), lambda i, tl: (tl[i], 0, 0)),
            pl.BlockSpec((1, 1, D_OUT), lambda i, tl: (tl[i], 0, 0)),
        ],
        out_specs=pl.BlockSpec((BT_MM, D_OUT), lambda i, tl: (i, 0)),
    )
    return pl.pallas_call(
        _mm_body,
        grid_spec=grid_spec,
        out_shape=jax.ShapeDtypeStruct((S, D_OUT), jnp.float32),
    )(tile_leaf, xs_sorted, leaf_W, leaf_b.reshape(N_LEAF, 1, D_OUT))


def kernel(xs, weights, biases, masks, leaf_W, leaf_b):
    mperm = jnp.concatenate([masks[:, 0::2], masks[:, 1::2]], axis=1)
    b2 = biases.reshape(N_INT, 1)
    tri = jnp.triu(jnp.ones((BT_B, BT_B), jnp.float32), 1)
    pos3, tl3 = _ab(xs, weights, b2, mperm, tri)
    pos = pos3.reshape(T)
    tile_leaf = tl3.reshape(TL_PAD)
    xs_sorted = _sc_permute(xs, pos, S, scatter=True)
    out_sorted = _grouped_mm(tile_leaf, xs_sorted, leaf_W, leaf_b)
    return _sc_permute(out_sorted, pos, T, scatter=False)


# speedup vs baseline: 1.1782x; 1.0092x over previous
import functools

import jax
import jax.numpy as jnp
from jax import lax
from jax.experimental import pallas as pl
from jax.experimental.pallas import tpu as pltpu
from jax.experimental.pallas import tpu_sc as plsc

T = 8192
D_MODEL = 1024
D_OUT = 1024
N_LEAF = 16
N_INT = 15

BT_A = 512
NT_A = T // BT_A
BT_B = 512
NT_B = T // BT_B
BT_MM = 256
S = T + N_LEAF * BT_MM
NT_MM = S // BT_MM
TL_PAD = 128


def _ab_body(x_ref, w_ref, b_ref, m_ref, tri_ref, pos_ref, tl_ref, leafbuf, scr):
    p = pl.program_id(0)
    i = pl.program_id(1)

    @pl.when(p == 0)
    def _():
        x = x_ref[...]
        logits = lax.dot_general(w_ref[...], x, (((1,), (1,)), ((), ())),
                                 preferred_element_type=jnp.float32) + b_ref[...]
        lp = jnp.concatenate(
            [jax.nn.log_sigmoid(-logits), jax.nn.log_sigmoid(logits)], axis=0)
        log_probs = lax.dot_general(m_ref[...], lp, (((1,), (0,)), ((), ())),
                                    preferred_element_type=jnp.float32)
        row = lax.broadcasted_iota(jnp.int32, (N_LEAF, BT_A), 0)
        mx = jnp.max(log_probs, axis=0, keepdims=True)
        leaf = jnp.min(jnp.where(log_probs == mx, row, N_LEAF), axis=0,
                       keepdims=True)
        leafbuf[pl.ds(i, 1), :] = leaf
        onehot = (leaf == row).astype(jnp.float32)
        tile_cnt = jnp.sum(onehot, axis=1, keepdims=True).astype(jnp.int32)

        @pl.when(i == 0)
        def _():
            scr[...] = jnp.zeros_like(scr)

        scr[0:N_LEAF, 0:1] = scr[0:N_LEAF, 0:1] + tile_cnt

    @pl.when((p == 1) & (i == 0))
    def _():
        cnt = scr[0:N_LEAF, 0:1]
        padded = ((cnt + BT_MM - 1) // BT_MM) * BT_MM
        rowj = lax.broadcasted_iota(jnp.int32, (N_LEAF, N_LEAF), 0)
        colk = lax.broadcasted_iota(jnp.int32, (N_LEAF, N_LEAF), 1)
        low = (colk < rowj).astype(jnp.float32)
        gs = lax.dot_general(low, padded.astype(jnp.float32),
                             (((1,), (0,)), ((), ())),
                             preferred_element_type=jnp.float32
                             ).astype(jnp.int32)
        scr[0:N_LEAF, 1:2] = gs
        ends = gs + padded
        rr = lax.broadcasted_iota(jnp.int32, (N_LEAF, TL_PAD), 1) * BT_MM
        tl = jnp.sum((rr >= ends).astype(jnp.int32), axis=0, keepdims=True)
        tl_ref[0, 0, :] = jnp.minimum(tl, N_LEAF - 1)[0]
        scr[0:N_LEAF, 0:1] = jnp.zeros((N_LEAF, 1), jnp.int32)

    @pl.when(p == 1)
    def _():
        lf = leafbuf[pl.ds(i, 1), :]
        row = lax.broadcasted_iota(jnp.int32, (N_LEAF, BT_B), 0)
        onehot = (lf == row).astype(jnp.float32)
        tile_cnt = jnp.sum(onehot, axis=1, keepdims=True).astype(jnp.int32)
        prev = scr[0:N_LEAF, 0:1]
        gs = scr[0:N_LEAF, 1:2]
        rank = lax.dot_general(onehot, tri_ref[...], (((1,), (0,)), ((), ())),
                               preferred_element_type=jnp.float32)
        basef = (gs + prev).astype(jnp.float32)
        posv = jnp.sum(onehot * (basef + rank), axis=0, keepdims=True)
        pos_ref[0, 0, :] = posv.astype(jnp.int32)[0]
        scr[0:N_LEAF, 0:1] = prev + tile_cnt


def _ab(xs, weights, b2, mperm, tri):
    return pl.pallas_call(
        _ab_body,
        grid=(2, NT_B),
        in_specs=[
            pl.BlockSpec((BT_A, D_MODEL),
                         lambda p, i: (jnp.where(p == 0, i, 0), 0)),
            pl.BlockSpec((N_INT, D_MODEL), lambda p, i: (0, 0)),
            pl.BlockSpec((N_INT, 1), lambda p, i: (0, 0)),
            pl.BlockSpec((N_LEAF, 2 * N_INT), lambda p, i: (0, 0)),
            pl.BlockSpec((BT_B, BT_B), lambda p, i: (0, 0)),
        ],
        out_specs=[
            pl.BlockSpec((1, 1, BT_B), lambda p, i: (i, 0, 0)),
            pl.BlockSpec((1, 1, TL_PAD), lambda p, i: (0, 0, 0)),
        ],
        out_shape=[
            jax.ShapeDtypeStruct((NT_B, 1, BT_B), jnp.int32),
            jax.ShapeDtypeStruct((1, 1, TL_PAD), jnp.int32),
        ],
        scratch_shapes=[pltpu.VMEM((NT_B, BT_B), jnp.int32),
                        pltpu.VMEM((N_LEAF, 128), jnp.int32)],
    )(xs, weights, b2, mperm, tri)


_NC = 2
_NS = 16
_NW = _NC * _NS
_PER_W = T // _NW
_SC_CHUNK = 32
_NCH = _PER_W // _SC_CHUNK


def _sc_permute(src, pos, out_rows, scatter):
    mesh = plsc.VectorSubcoreMesh(core_axis_name="c", subcore_axis_name="s")

    @functools.partial(
        pl.kernel, mesh=mesh,
        out_type=jax.ShapeDtypeStruct((out_rows, D_OUT), jnp.float32),
        scratch_types=(
            [pltpu.VMEM((_SC_CHUNK,), jnp.int32) for _ in range(_NCH)]
            + [pltpu.VMEM((_SC_CHUNK, D_OUT), jnp.float32)
               for _ in range(3)]
            + [pltpu.SemaphoreType.DMA for _ in range(7)]),
    )
    def k(src_hbm, pos_hbm, out_hbm, i0, i1, i2, i3, i4, i5, i6, i7,
          b0, b1, b2, sem_idx, si0, si1, si2, so0, so1, so2):
        wid = lax.axis_index("s") * _NC + lax.axis_index("c")
        base = wid * _PER_W
        idx = [i0, i1, i2, i3, i4, i5, i6, i7]
        bufs = [b0, b1, b2]
        sin = [si0, si1, si2]
        sout = [so0, so1, so2]
        hidx = [pltpu.async_copy(
                    pos_hbm.at[pl.ds(base + c * _SC_CHUNK, _SC_CHUNK)],
                    idx[c], sem_idx) for c in range(_NCH)]
        for h in hidx:
            h.wait()

        def start_in(c):
            lin = pl.ds(base + c * _SC_CHUNK, _SC_CHUNK)
            if scatter:
                return pltpu.async_copy(src_hbm.at[lin], bufs[c % 3],
                                        sin[c % 3])
            return pltpu.async_copy(src_hbm.at[idx[c]], bufs[c % 3],
                                    sin[c % 3])

        def start_out(c):
            lin = pl.ds(base + c * _SC_CHUNK, _SC_CHUNK)
            if scatter:
                return pltpu.async_copy(bufs[c % 3], out_hbm.at[idx[c]],
                                        sout[c % 3])
            return pltpu.async_copy(bufs[c % 3], out_hbm.at[lin], sout[c % 3])

        ins = [None] * _NCH
        outs = [None] * _NCH
        ins[0] = start_in(0)
        ins[1] = start_in(1)
        ins[2] = start_in(2)
        for c in range(_NCH):
            ins[c].wait()
            outs[c] = start_out(c)
            if c + 3 < _NCH:
                outs[c].wait()
                ins[c + 3] = start_in(c + 3)
        outs[_NCH - 3].wait()
        outs[_NCH - 2].wait()
        outs[_NCH - 1].wait()

    return k(src, pos)


def _mm_body(tl_ref, x_ref, w_ref, b_ref, o_ref):
    del tl_ref
    o_ref[...] = lax.dot_general(
        x_ref[...], w_ref[0], (((1,), (0,)), ((), ())),
        preferred_element_type=jnp.float32) + b_ref[0]


def _grouped_mm(tile_leaf, xs_sorted, leaf_W, leaf_b):
    grid_spec = pltpu.PrefetchScalarGridSpec(
        num_scalar_prefetch=1,
        grid=(NT_MM,),
        in_specs=[
            pl.BlockSpec((BT_MM, D_MODEL), lambda i, tl: (i, 0)),
            pl.BlockSpec((1, D_MODEL, D_OUT), lambda i, tl: (tl[i], 0, 0)),
            pl.BlockSpec((1, 1, D_OUT), lambda i, tl: (tl[i], 0, 0)),
        ],
        out_specs=pl.BlockSpec((BT_MM, D_OUT), lambda i, tl: (i, 0)),
    )
    return pl.pallas_call(
        _mm_body,
        grid_spec=grid_spec,
        out_shape=jax.ShapeDtypeStruct((S, D_OUT), jnp.float32),
    )(tile_leaf, xs_sorted, leaf_W, leaf_b.reshape(N_LEAF, 1, D_OUT))


def kernel(xs, weights, biases, masks, leaf_W, leaf_b):
    mperm = jnp.concatenate([masks[:, 0::2], masks[:, 1::2]], axis=1)
    b2 = biases.reshape(N_INT, 1)
    tri = jnp.triu(jnp.ones((BT_B, BT_B), jnp.float32), 1)
    pos3, tl3 = _ab(xs, weights, b2, mperm, tri)
    pos = pos3.reshape(T)
    tile_leaf = tl3.reshape(TL_PAD)
    xs_sorted = _sc_permute(xs, pos, S, scatter=True)
    out_sorted = _grouped_mm(tile_leaf, xs_sorted, leaf_W, leaf_b)
    return _sc_permute(out_sorted, pos, T, scatter=False)
